# baseline (speedup 1.0000x reference)
import functools

import jax
import jax.numpy as jnp
from jax import lax
from jax.experimental import pallas as pl
from jax.experimental.pallas import tpu as pltpu
from jax.experimental.pallas import tpu_sc as plsc

D = 128
N = 10000
NPAD = 10240
E = 320000
G = 256
NPRED = 100
PPAD = 104
NC, NS = 2, 16
NW = NC * NS
EW = 10240
EPAD = EW * NW
K = 128
NCHUNK = EW // K
NPAIR = NCHUNK // 2
ROWS_PER_TILE = NPAD // NS
BN = 1024
NBLK = NPAD // BN

def _relu(v):
    return jnp.maximum(v, 0.0)


@functools.lru_cache(maxsize=None)
def _make_edge_pass(with_p, k, ew0=EW):
    ew1 = 2 * EW - ew0
    np0, np1 = ew0 // (2 * k), ew1 // (2 * k)
    scratch = [
        pltpu.VMEM_SHARED((NPAD, D), jnp.float32),
    ]
    for _ in range(2):
        scratch += [
            pltpu.VMEM((k,), jnp.int32),
            pltpu.VMEM((k,), jnp.int32),
            pltpu.VMEM((k, D), jnp.float32),
            pltpu.SemaphoreType.DMA,
            pltpu.SemaphoreType.DMA,
            pltpu.SemaphoreType.DMA,
        ]
        if with_p:
            scratch += [
                pltpu.VMEM((k,), jnp.int32),
                pltpu.VMEM((k, D), jnp.float32),
                pltpu.SemaphoreType.DMA,
            ]

    def body(*refs):
        nh = 5 if with_p else 3
        hbm = refs[:nh]
        out_h = refs[nh]
        sc = list(refs[nh + 1:])
        aggr_sh = sc.pop(0)
        nper = len(sc) // 2
        bufs = [sc[:nper], sc[nper:]]
        if with_p:
            attr_h, dst_h, t_h, src_h, p_h = hbm
        else:
            attr_h, dst_h, t_h = hbm
        c = lax.axis_index("c")
        s = lax.axis_index("s")
        npair = jnp.where(c == 0, np0, np1)
        ebase = jnp.where(c == 0, s * ew0, NS * ew0 + s * ew1)

        def unpack(b):
            if with_p:
                aidx, didx, rows, sem_i, sem_g, sem_s, sidx, prow, sem_p = \
                    bufs[b]
            else:
                aidx, didx, rows, sem_i, sem_g, sem_s = bufs[b]
                sidx = prow = sem_p = None
            return aidx, didx, rows, sem_i, sem_g, sem_s, sidx, prow, sem_p

        def start_idx(b, base):
            aidx, didx, _, sem_i, _, _, sidx, _, _ = unpack(b)
            pltpu.async_copy(attr_h.at[pl.ds(base, k)], aidx, sem_i)
            pltpu.async_copy(dst_h.at[pl.ds(base, k)], didx, sem_i)
            if with_p:
                pltpu.async_copy(src_h.at[pl.ds(base, k)], sidx, sem_i)

        def wait_idx_start_gathers(b, base):
            aidx, didx, rows, sem_i, sem_g, _, sidx, prow, sem_p = unpack(b)
            pltpu.make_async_copy(attr_h.at[pl.ds(base, k)], aidx,
                                  sem_i).wait()
            pltpu.make_async_copy(dst_h.at[pl.ds(base, k)], didx,
                                  sem_i).wait()
            if with_p:
                pltpu.make_async_copy(src_h.at[pl.ds(base, k)], sidx,
                                      sem_i).wait()
                pltpu.async_copy(p_h.at[sidx], prow, sem_p)
            pltpu.async_copy(t_h.at[aidx], rows, sem_g)

        def wait_gathers(b):
            aidx, _, rows, _, sem_g, _, sidx, prow, sem_p = unpack(b)
            pltpu.make_async_copy(t_h.at[aidx], rows, sem_g).wait()
            if with_p:
                pltpu.make_async_copy(p_h.at[sidx], prow, sem_p).wait()

        _, _, rows0, _, _, _, _, _, _ = unpack(0)

        def zrow(r, carry):
            for cc in range(D // 16):
                rows0[r, pl.ds(cc * 16, 16)] = jnp.zeros((16,), jnp.float32)
            return carry
        lax.fori_loop(0, k, zrow, 0)
        row0 = s * ROWS_PER_TILE
        for q in range(ROWS_PER_TILE // k):
            pltpu.sync_copy(rows0, aggr_sh.at[pl.ds(row0 + q * k, k)])
        plsc.subcore_barrier()
        start_idx(0, ebase)
        wait_idx_start_gathers(0, ebase)

        def pair(t, carry):
            for b in range(2):
                j2 = 2 * t + b
                nb = 1 - b
                (aidx, didx, rows, sem_i, sem_g, sem_s,
                 sidx, prow, sem_p) = unpack(b)
                n_bufs = unpack(nb)
                nbase = ebase + (j2 + 1) * k

                def wait_prev_scat():
                    pltpu.make_async_copy(
                        n_bufs[2], aggr_sh.at[n_bufs[1]], n_bufs[5]).wait()

                def launch():
                    wait_idx_start_gathers(nb, nbase)

                if b == 1:
                    wait_prev_scat()
                else:
                    pl.when(t > 0)(wait_prev_scat)
                if b == 0:
                    start_idx(nb, nbase)
                else:
                    pl.when(t < npair - 1)(lambda: start_idx(nb, nbase))
                wait_gathers(b)
                if with_p:
                    def rbody(r4, rc):
                        for u in range(4):
                            r = 4 * r4 + u
                            for cc in range(D // 16):
                                sl = pl.ds(cc * 16, 16)
                                rows[r, sl] = _relu(rows[r, sl]
                                                    + prow[r, sl])
                        return rc
                    lax.fori_loop(0, k // 4, rbody, 0)
                if b == 0:
                    launch()
                else:
                    pl.when(t < npair - 1)(launch)
                pltpu.async_copy(rows, aggr_sh.at[didx], sem_s, add=True)
            return carry
        lax.fori_loop(0, npair, pair, 0)
        b1 = unpack(1)
        pltpu.make_async_copy(b1[2], aggr_sh.at[b1[1]], b1[5]).wait()
        plsc.subcore_barrier()
        pltpu.sync_copy(aggr_sh.at[pl.ds(row0, ROWS_PER_TILE)],
                        out_h.at[c, pl.ds(row0, ROWS_PER_TILE)])

    mesh = plsc.VectorSubcoreMesh(
        core_axis_name="c", subcore_axis_name="s",
        num_cores=NC, num_subcores=NS)
    return pl.kernel(
        body,
        out_type=jax.ShapeDtypeStruct((NC, NPAD, D), jnp.float32),
        mesh=mesh,
        scratch_types=scratch,
    )


def _edge_pass_l0(*args):
    return _make_edge_pass(False, 128, 17408)(*args)


def _edge_pass_l1(*args):
    return _make_edge_pass(True, 80, 17440)(*args)


def _prep_tables(relp, w0b, b0, w1b, b1):
    def body(r_ref, w0_ref, b0_ref, w1_ref, b1_ref, t0_ref, t1_ref):
        r = r_ref[...]
        t0_ref[...] = _relu(
            jnp.dot(r, w0_ref[...], preferred_element_type=jnp.float32)
            + b0_ref[...])
        t1_ref[...] = (
            jnp.dot(r, w1_ref[...], preferred_element_type=jnp.float32)
            + b1_ref[...])
    return pl.pallas_call(
        body,
        out_shape=(jax.ShapeDtypeStruct((PPAD, D), jnp.float32),
                   jax.ShapeDtypeStruct((PPAD, D), jnp.float32)),
    )(relp, w0b, b0, w1b, b1)


def _node_update(parts, h, wa, wb, cb, gw, gb, batch_col):
    with_h = h is not None

    def body(*refs):
        if with_h:
            (parts_ref, h_ref, wa_ref, wb_ref, cb_ref, gw_ref, gb_ref,
             bi_ref, hn_ref, gate_ref, gmax_ref) = refs
        else:
            (parts_ref, wb_ref, cb_ref, gw_ref, gb_ref,
             bi_ref, hn_ref, gate_ref, gmax_ref) = refs
        i = pl.program_id(0)
        aggr = parts_ref[0] + parts_ref[1]
        z = jnp.dot(aggr, wb_ref[...], preferred_element_type=jnp.float32)
        z = z + cb_ref[...]
        if with_h:
            hb = h_ref[...]
            z = z + jnp.dot(hb, wa_ref[...],
                            preferred_element_type=jnp.float32)
            hn = hb + _relu(z)
        else:
            hn = _relu(z)
        hn_ref[...] = hn
        gate = (jnp.dot(hn, gw_ref[...], preferred_element_type=jnp.float32)
                + gb_ref[...])
        gate_ref[...] = gate
        mask = bi_ref[...] == lax.broadcasted_iota(jnp.int32, (BN, G), 1)
        bmax = jnp.max(jnp.where(mask, gate, -jnp.inf), axis=0, keepdims=True)

        @pl.when(i == 0)
        def _():
            gmax_ref[...] = jnp.full((1, G), -jnp.inf, jnp.float32)
        gmax_ref[...] = jnp.maximum(gmax_ref[...], bmax)

    full = lambda shape: pl.BlockSpec(shape, lambda i: (0,) * len(shape))
    in_specs = [pl.BlockSpec((NC, BN, D), lambda i: (0, i, 0))]
    args = [parts]
    if with_h:
        in_specs.append(pl.BlockSpec((BN, D), lambda i: (i, 0)))
        args.append(h)
        in_specs.append(full((D, D)))
        args.append(wa)
    in_specs += [full((D, D)), full((1, D)), full((D, 1)), full((1, 1)),
                 pl.BlockSpec((BN, 1), lambda i: (i, 0))]
    args += [wb, cb, gw, gb, batch_col]
    return pl.pallas_call(
        body,
        grid=(NBLK,),
        in_specs=in_specs,
        out_specs=(pl.BlockSpec((BN, D), lambda i: (i, 0)),
                   pl.BlockSpec((BN, 1), lambda i: (i, 0)),
                   pl.BlockSpec((1, G), lambda i: (0, 0))),
        out_shape=(jax.ShapeDtypeStruct((NPAD, D), jnp.float32),
                   jax.ShapeDtypeStruct((NPAD, 1), jnp.float32),
                   jax.ShapeDtypeStruct((1, G), jnp.float32)),
    )(*args)


def _pool(hn, gate, gmax, batch_col, aw, ab, pw):
    with_p = pw is not None

    def body(*refs):
        if with_p:
            (h_ref, gate_ref, gmax_ref, bi_ref, aw_ref, ab_ref, pw_ref,
             glnum_ref, denom_ref, p_ref) = refs
        else:
            (h_ref, gate_ref, gmax_ref, bi_ref, aw_ref, ab_ref,
             glnum_ref, denom_ref) = refs
        i = pl.program_id(0)
        gm = gmax_ref[...]
        gmf = jnp.where(jnp.isfinite(gm), gm, 0.0)
        bi = bi_ref[...]
        mask = bi == lax.broadcasted_iota(jnp.int32, (BN, G), 1)
        ohf = mask.astype(jnp.float32)
        gnode = jnp.sum(jnp.where(mask, gmf, 0.0), axis=1, keepdims=True)
        egate = jnp.where(bi < G, jnp.exp(gate_ref[...] - gnode), 0.0)
        hb = h_ref[...]
        xt = (jnp.dot(hb, aw_ref[...], preferred_element_type=jnp.float32)
              + ab_ref[...])
        gn = lax.dot_general(ohf, egate * xt, (((0,), (0,)), ((), ())),
                             preferred_element_type=jnp.float32)
        dn = lax.dot_general(ohf, egate, (((0,), (0,)), ((), ())),
                             preferred_element_type=jnp.float32)

        @pl.when(i == 0)
        def _():
            glnum_ref[...] = jnp.zeros((G, D), jnp.float32)
            denom_ref[...] = jnp.zeros((G, 1), jnp.float32)
        glnum_ref[...] += gn
        denom_ref[...] += dn
        if with_p:
            p_ref[...] = jnp.dot(hb, pw_ref[...],
                                 preferred_element_type=jnp.float32)

    full = lambda shape: pl.BlockSpec(shape, lambda i: (0,) * len(shape))
    in_specs = [pl.BlockSpec((BN, D), lambda i: (i, 0)),
                pl.BlockSpec((BN, 1), lambda i: (i, 0)),
                full((1, G)),
                pl.BlockSpec((BN, 1), lambda i: (i, 0)),
                full((D, D)), full((1, D))]
    args = [hn, gate, gmax, batch_col, aw, ab]
    out_specs = [pl.BlockSpec((G, D), lambda i: (0, 0)),
                 pl.BlockSpec((G, 1), lambda i: (0, 0))]
    out_shape = [jax.ShapeDtypeStruct((G, D), jnp.float32),
                 jax.ShapeDtypeStruct((G, 1), jnp.float32)]
    if with_p:
        in_specs.append(full((D, D)))
        args.append(pw)
        out_specs.append(pl.BlockSpec((BN, D), lambda i: (i, 0)))
        out_shape.append(jax.ShapeDtypeStruct((NPAD, D), jnp.float32))
    return pl.pallas_call(
        body,
        grid=(NBLK,),
        in_specs=in_specs,
        out_specs=tuple(out_specs),
        out_shape=tuple(out_shape),
    )(*args)


def _global_update(glnum, denom, g, wa, wb, glb):
    with_g = g is not None

    def body(*refs):
        if with_g:
            gl_ref, dn_ref, g_ref, wa_ref, wb_ref, b_ref, out_ref = refs
        else:
            gl_ref, dn_ref, wa_ref, b_ref, out_ref = refs
        gl = gl_ref[...] / (dn_ref[...] + 1e-16)
        z = jnp.dot(gl, wa_ref[...], preferred_element_type=jnp.float32)
        z = z + b_ref[...]
        if with_g:
            gprev = g_ref[...]
            z = z + jnp.dot(gprev, wb_ref[...],
                            preferred_element_type=jnp.float32)
            out_ref[...] = gprev + _relu(z)
        else:
            out_ref[...] = _relu(z)

    args = [glnum, denom] + ([g] if with_g else []) + \
        ([wa, wb] if with_g else [wa]) + [glb]
    return pl.pallas_call(
        body,
        out_shape=jax.ShapeDtypeStruct((G, D), jnp.float32),
    )(*args)


def kernel(params, x, edge_index, edge_attr, batch_idx):
    src = edge_index[0].astype(jnp.int32)
    dst = edge_index[1].astype(jnp.int32)
    attr = edge_attr.astype(jnp.int32)
    batch = batch_idx.astype(jnp.int32)

    pad_e = EPAD - E
    src_p = jnp.concatenate([src, jnp.zeros((pad_e,), jnp.int32)])
    dst_p = jnp.concatenate([dst, jnp.full((pad_e,), N, jnp.int32)])
    attr_p = jnp.concatenate([attr, jnp.zeros((pad_e,), jnp.int32)])
    batch_col = jnp.concatenate(
        [batch, jnp.full((NPAD - N,), G, jnp.int32)]).reshape(NPAD, 1)

    relp = jnp.pad(params['rel_table'], ((0, PPAD - NPRED), (0, 0)))
    row = lambda b: b.reshape(1, -1)

    msg_W0, msg_b0 = params['msg_W0'], params['msg_b0']
    msg_W1, msg_b1 = params['msg_W1'], params['msg_b1']
    t0r, t1 = _prep_tables(relp, msg_W0[D:], row(msg_b0),
                           msg_W1[D:], row(msg_b1))

    parts0 = _edge_pass_l0(attr_p, dst_p, t0r)
    h1, gate0, gmax0 = _node_update(
        parts0, None, None, params['comb_W0'][D:], row(params['comb_b0']),
        params['gate_W0'], row(params['gate_b0']), batch_col)
    glnum0, denom0, p1 = _pool(h1, gate0, gmax0, batch_col,
                               params['attn_W0'], row(params['attn_b0']),
                               msg_W1[:D])
    g1 = _global_update(glnum0, denom0, None,
                        params['glob_W0'][:D], None, row(params['glob_b0']))

    parts1 = _edge_pass_l1(attr_p, dst_p, t1, src_p, p1)
    h2, gate1, gmax1 = _node_update(
        parts1, h1, params['comb_W1'][:D], params['comb_W1'][D:],
        row(params['comb_b1']), params['gate_W1'], row(params['gate_b1']),
        batch_col)
    glnum1, denom1 = _pool(h2, gate1, gmax1, batch_col,
                           params['attn_W1'], row(params['attn_b1']), None)
    g2 = _global_update(glnum1, denom1, g1,
                        params['glob_W1'][:D], params['glob_W1'][D:],
                        row(params['glob_b1']))

    return h2[:N], g2


# speedup vs baseline: 1.0143x; 1.0143x over previous
import functools

import jax
import jax.numpy as jnp
from jax import lax
from jax.experimental import pallas as pl
from jax.experimental.pallas import tpu as pltpu
from jax.experimental.pallas import tpu_sc as plsc

D = 128
N = 10000
NPAD = 10240
E = 320000
G = 256
NPRED = 100
PPAD = 104
NC, NS = 2, 16
NW = NC * NS
EW = 10240
EPAD = EW * NW
K = 128
NCHUNK = EW // K
NPAIR = NCHUNK // 2
ROWS_PER_TILE = NPAD // NS
BN = 1024
NBLK = NPAD // BN

def _relu(v):
    return jnp.maximum(v, 0.0)


@functools.lru_cache(maxsize=None)
def _make_edge_pass(with_p, k, ew0=EW):
    ew1 = 2 * EW - ew0
    np0, np1 = ew0 // (2 * k), ew1 // (2 * k)
    scratch = [
        pltpu.VMEM_SHARED((NPAD, D), jnp.float32),
    ]
    for _ in range(2):
        scratch += [
            pltpu.VMEM((k,), jnp.int32),
            pltpu.VMEM((k,), jnp.int32),
            pltpu.VMEM((k, D), jnp.float32),
            pltpu.SemaphoreType.DMA,
            pltpu.SemaphoreType.DMA,
            pltpu.SemaphoreType.DMA,
        ]
        if with_p:
            scratch += [
                pltpu.VMEM((k,), jnp.int32),
                pltpu.VMEM((k, D), jnp.float32),
                pltpu.SemaphoreType.DMA,
            ]

    def body(*refs):
        nh = 5 if with_p else 3
        hbm = refs[:nh]
        out_h = refs[nh]
        sc = list(refs[nh + 1:])
        aggr_sh = sc.pop(0)
        nper = len(sc) // 2
        bufs = [sc[:nper], sc[nper:]]
        if with_p:
            attr_h, dst_h, t_h, src_h, p_h = hbm
        else:
            attr_h, dst_h, t_h = hbm
        c = lax.axis_index("c")
        s = lax.axis_index("s")
        npair = jnp.where(c == 0, np0, np1)
        ebase = jnp.where(c == 0, s * ew0, NS * ew0 + s * ew1)

        def unpack(b):
            if with_p:
                aidx, didx, rows, sem_i, sem_g, sem_s, sidx, prow, sem_p = \
                    bufs[b]
            else:
                aidx, didx, rows, sem_i, sem_g, sem_s = bufs[b]
                sidx = prow = sem_p = None
            return aidx, didx, rows, sem_i, sem_g, sem_s, sidx, prow, sem_p

        def start_idx(b, base):
            aidx, didx, _, sem_i, _, _, sidx, _, _ = unpack(b)
            pltpu.async_copy(attr_h.at[pl.ds(base, k)], aidx, sem_i)
            pltpu.async_copy(dst_h.at[pl.ds(base, k)], didx, sem_i)
            if with_p:
                pltpu.async_copy(src_h.at[pl.ds(base, k)], sidx, sem_i)

        def wait_idx_start_gathers(b, base):
            aidx, didx, rows, sem_i, sem_g, _, sidx, prow, sem_p = unpack(b)
            pltpu.make_async_copy(attr_h.at[pl.ds(base, k)], aidx,
                                  sem_i).wait()
            pltpu.make_async_copy(dst_h.at[pl.ds(base, k)], didx,
                                  sem_i).wait()
            if with_p:
                pltpu.make_async_copy(src_h.at[pl.ds(base, k)], sidx,
                                      sem_i).wait()
                pltpu.async_copy(p_h.at[sidx], prow, sem_p)
            pltpu.async_copy(t_h.at[aidx], rows, sem_g)

        def wait_gathers(b):
            aidx, _, rows, _, sem_g, _, sidx, prow, sem_p = unpack(b)
            pltpu.make_async_copy(t_h.at[aidx], rows, sem_g).wait()
            if with_p:
                pltpu.make_async_copy(p_h.at[sidx], prow, sem_p).wait()

        _, _, rows0, _, _, _, _, _, _ = unpack(0)

        def zrow(r, carry):
            for cc in range(D // 16):
                rows0[r, pl.ds(cc * 16, 16)] = jnp.zeros((16,), jnp.float32)
            return carry
        lax.fori_loop(0, k, zrow, 0)
        row0 = s * ROWS_PER_TILE
        for q in range(ROWS_PER_TILE // k):
            pltpu.sync_copy(rows0, aggr_sh.at[pl.ds(row0 + q * k, k)])
        plsc.subcore_barrier()
        start_idx(0, ebase)
        wait_idx_start_gathers(0, ebase)

        def pair(t, carry):
            for b in range(2):
                j2 = 2 * t + b
                nb = 1 - b
                (aidx, didx, rows, sem_i, sem_g, sem_s,
                 sidx, prow, sem_p) = unpack(b)
                n_bufs = unpack(nb)
                nbase = ebase + (j2 + 1) * k

                def wait_prev_scat():
                    pltpu.make_async_copy(
                        n_bufs[2], aggr_sh.at[n_bufs[1]], n_bufs[5]).wait()

                def launch():
                    wait_idx_start_gathers(nb, nbase)

                if b == 1:
                    wait_prev_scat()
                else:
                    pl.when(t > 0)(wait_prev_scat)
                if b == 0:
                    start_idx(nb, nbase)
                else:
                    pl.when(t < npair - 1)(lambda: start_idx(nb, nbase))
                wait_gathers(b)
                if with_p:
                    def rbody(r4, rc):
                        for u in range(4):
                            r = 4 * r4 + u
                            for cc in range(D // 16):
                                sl = pl.ds(cc * 16, 16)
                                rows[r, sl] = _relu(rows[r, sl]
                                                    + prow[r, sl])
                        return rc
                    lax.fori_loop(0, k // 4, rbody, 0)
                if b == 0:
                    launch()
                else:
                    pl.when(t < npair - 1)(launch)
                pltpu.async_copy(rows, aggr_sh.at[didx], sem_s, add=True)
            return carry
        lax.fori_loop(0, npair, pair, 0)
        b1 = unpack(1)
        pltpu.make_async_copy(b1[2], aggr_sh.at[b1[1]], b1[5]).wait()
        plsc.subcore_barrier()
        pltpu.sync_copy(aggr_sh.at[pl.ds(row0, ROWS_PER_TILE)],
                        out_h.at[c, pl.ds(row0, ROWS_PER_TILE)])

    mesh = plsc.VectorSubcoreMesh(
        core_axis_name="c", subcore_axis_name="s",
        num_cores=NC, num_subcores=NS)
    return pl.kernel(
        body,
        out_type=jax.ShapeDtypeStruct((NC, NPAD, D), jnp.float32),
        mesh=mesh,
        scratch_types=scratch,
    )


def _edge_pass_l0(*args):
    return _make_edge_pass(False, 128, 16128)(*args)


def _edge_pass_l1(*args):
    return _make_edge_pass(True, 80, 16000)(*args)


def _prep_tables(relp, w0b, b0, w1b, b1):
    def body(r_ref, w0_ref, b0_ref, w1_ref, b1_ref, t0_ref, t1_ref):
        r = r_ref[...]
        t0_ref[...] = _relu(
            jnp.dot(r, w0_ref[...], preferred_element_type=jnp.float32)
            + b0_ref[...])
        t1_ref[...] = (
            jnp.dot(r, w1_ref[...], preferred_element_type=jnp.float32)
            + b1_ref[...])
    return pl.pallas_call(
        body,
        out_shape=(jax.ShapeDtypeStruct((PPAD, D), jnp.float32),
                   jax.ShapeDtypeStruct((PPAD, D), jnp.float32)),
    )(relp, w0b, b0, w1b, b1)


def _node_update(parts, h, wa, wb, cb, gw, gb, batch_col):
    with_h = h is not None

    def body(*refs):
        if with_h:
            (parts_ref, h_ref, wa_ref, wb_ref, cb_ref, gw_ref, gb_ref,
             bi_ref, hn_ref, gate_ref, gmax_ref) = refs
        else:
            (parts_ref, wb_ref, cb_ref, gw_ref, gb_ref,
             bi_ref, hn_ref, gate_ref, gmax_ref) = refs
        i = pl.program_id(0)
        aggr = parts_ref[0] + parts_ref[1]
        z = jnp.dot(aggr, wb_ref[...], preferred_element_type=jnp.float32)
        z = z + cb_ref[...]
        if with_h:
            hb = h_ref[...]
            z = z + jnp.dot(hb, wa_ref[...],
                            preferred_element_type=jnp.float32)
            hn = hb + _relu(z)
        else:
            hn = _relu(z)
        hn_ref[...] = hn
        gate = (jnp.dot(hn, gw_ref[...], preferred_element_type=jnp.float32)
                + gb_ref[...])
        gate_ref[...] = gate
        mask = bi_ref[...] == lax.broadcasted_iota(jnp.int32, (BN, G), 1)
        bmax = jnp.max(jnp.where(mask, gate, -jnp.inf), axis=0, keepdims=True)

        @pl.when(i == 0)
        def _():
            gmax_ref[...] = jnp.full((1, G), -jnp.inf, jnp.float32)
        gmax_ref[...] = jnp.maximum(gmax_ref[...], bmax)

    full = lambda shape: pl.BlockSpec(shape, lambda i: (0,) * len(shape))
    in_specs = [pl.BlockSpec((NC, BN, D), lambda i: (0, i, 0))]
    args = [parts]
    if with_h:
        in_specs.append(pl.BlockSpec((BN, D), lambda i: (i, 0)))
        args.append(h)
        in_specs.append(full((D, D)))
        args.append(wa)
    in_specs += [full((D, D)), full((1, D)), full((D, 1)), full((1, 1)),
                 pl.BlockSpec((BN, 1), lambda i: (i, 0))]
    args += [wb, cb, gw, gb, batch_col]
    return pl.pallas_call(
        body,
        grid=(NBLK,),
        in_specs=in_specs,
        out_specs=(pl.BlockSpec((BN, D), lambda i: (i, 0)),
                   pl.BlockSpec((BN, 1), lambda i: (i, 0)),
                   pl.BlockSpec((1, G), lambda i: (0, 0))),
        out_shape=(jax.ShapeDtypeStruct((NPAD, D), jnp.float32),
                   jax.ShapeDtypeStruct((NPAD, 1), jnp.float32),
                   jax.ShapeDtypeStruct((1, G), jnp.float32)),
    )(*args)


def _pool(hn, gate, gmax, batch_col, aw, ab, pw, g, gwa, gwb, glb):
    with_p = pw is not None
    with_g = g is not None

    def body(*refs):
        refs = list(refs)
        h_ref, gate_ref, gmax_ref, bi_ref, aw_ref, ab_ref = refs[:6]
        rest = refs[6:]
        pw_ref = rest.pop(0) if with_p else None
        g_ref = rest.pop(0) if with_g else None
        gwa_ref, gwb_ref, glb_ref = rest[:3]
        outs = rest[3:]
        glnum_ref, denom_ref, gout_ref = outs[:3]
        p_ref = outs[3] if with_p else None
        i = pl.program_id(0)
        gm = gmax_ref[...]
        gmf = jnp.where(jnp.isfinite(gm), gm, 0.0)
        bi = bi_ref[...]
        mask = bi == lax.broadcasted_iota(jnp.int32, (BN, G), 1)
        ohf = mask.astype(jnp.float32)
        gnode = jnp.sum(jnp.where(mask, gmf, 0.0), axis=1, keepdims=True)
        egate = jnp.where(bi < G, jnp.exp(gate_ref[...] - gnode), 0.0)
        hb = h_ref[...]
        xt = (jnp.dot(hb, aw_ref[...], preferred_element_type=jnp.float32)
              + ab_ref[...])
        gn = lax.dot_general(ohf, egate * xt, (((0,), (0,)), ((), ())),
                             preferred_element_type=jnp.float32)
        dn = lax.dot_general(ohf, egate, (((0,), (0,)), ((), ())),
                             preferred_element_type=jnp.float32)

        @pl.when(i == 0)
        def _():
            glnum_ref[...] = jnp.zeros((G, D), jnp.float32)
            denom_ref[...] = jnp.zeros((G, 1), jnp.float32)
        glnum_ref[...] += gn
        denom_ref[...] += dn
        if with_p:
            p_ref[...] = jnp.dot(hb, pw_ref[...],
                                 preferred_element_type=jnp.float32)

        @pl.when(i == NBLK - 1)
        def _():
            gl = glnum_ref[...] / (denom_ref[...] + 1e-16)
            z = jnp.dot(gl, gwa_ref[...], preferred_element_type=jnp.float32)
            z = z + glb_ref[...]
            if with_g:
                gprev = g_ref[...]
                z = z + jnp.dot(gprev, gwb_ref[...],
                                preferred_element_type=jnp.float32)
                gout_ref[...] = gprev + _relu(z)
            else:
                gout_ref[...] = _relu(z)

    full = lambda shape: pl.BlockSpec(shape, lambda i: (0,) * len(shape))
    in_specs = [pl.BlockSpec((BN, D), lambda i: (i, 0)),
                pl.BlockSpec((BN, 1), lambda i: (i, 0)),
                full((1, G)),
                pl.BlockSpec((BN, 1), lambda i: (i, 0)),
                full((D, D)), full((1, D))]
    args = [hn, gate, gmax, batch_col, aw, ab]
    if with_p:
        in_specs.append(full((D, D)))
        args.append(pw)
    if with_g:
        in_specs.append(full((G, D)))
        args.append(g)
    in_specs += [full((D, D)), full((D, D)) if with_g else full((D, D)),
                 full((1, D))]
    args += [gwa, gwb if with_g else gwa, glb]
    out_specs = [pl.BlockSpec((G, D), lambda i: (0, 0)),
                 pl.BlockSpec((G, 1), lambda i: (0, 0)),
                 pl.BlockSpec((G, D), lambda i: (0, 0))]
    out_shape = [jax.ShapeDtypeStruct((G, D), jnp.float32),
                 jax.ShapeDtypeStruct((G, 1), jnp.float32),
                 jax.ShapeDtypeStruct((G, D), jnp.float32)]
    if with_p:
        out_specs.append(pl.BlockSpec((BN, D), lambda i: (i, 0)))
        out_shape.append(jax.ShapeDtypeStruct((NPAD, D), jnp.float32))
    res = pl.pallas_call(
        body,
        grid=(NBLK,),
        in_specs=in_specs,
        out_specs=tuple(out_specs),
        out_shape=tuple(out_shape),
    )(*args)
    return (res[2], res[3]) if with_p else (res[2],)


def kernel(params, x, edge_index, edge_attr, batch_idx):
    src = edge_index[0].astype(jnp.int32)
    dst = edge_index[1].astype(jnp.int32)
    attr = edge_attr.astype(jnp.int32)
    batch = batch_idx.astype(jnp.int32)

    pad_e = EPAD - E
    src_p = jnp.concatenate([src, jnp.zeros((pad_e,), jnp.int32)])
    dst_p = jnp.concatenate([dst, jnp.full((pad_e,), N, jnp.int32)])
    attr_p = jnp.concatenate([attr, jnp.zeros((pad_e,), jnp.int32)])
    batch_col = jnp.concatenate(
        [batch, jnp.full((NPAD - N,), G, jnp.int32)]).reshape(NPAD, 1)

    relp = jnp.pad(params['rel_table'], ((0, PPAD - NPRED), (0, 0)))
    row = lambda b: b.reshape(1, -1)

    msg_W0, msg_b0 = params['msg_W0'], params['msg_b0']
    msg_W1, msg_b1 = params['msg_W1'], params['msg_b1']
    t0r, t1 = _prep_tables(relp, msg_W0[D:], row(msg_b0),
                           msg_W1[D:], row(msg_b1))

    parts0 = _edge_pass_l0(attr_p, dst_p, t0r)
    h1, gate0, gmax0 = _node_update(
        parts0, None, None, params['comb_W0'][D:], row(params['comb_b0']),
        params['gate_W0'], row(params['gate_b0']), batch_col)
    g1, p1 = _pool(h1, gate0, gmax0, batch_col,
                   params['attn_W0'], row(params['attn_b0']), msg_W1[:D],
                   None, params['glob_W0'][:D], None,
                   row(params['glob_b0']))

    parts1 = _edge_pass_l1(attr_p, dst_p, t1, src_p, p1)
    h2, gate1, gmax1 = _node_update(
        parts1, h1, params['comb_W1'][:D], params['comb_W1'][D:],
        row(params['comb_b1']), params['gate_W1'], row(params['gate_b1']),
        batch_col)
    (g2,) = _pool(h2, gate1, gmax1, batch_col,
                  params['attn_W1'], row(params['attn_b1']), None,
                  g1, params['glob_W1'][:D], params['glob_W1'][D:],
                  row(params['glob_b1']))

    return h2[:N], g2


# speedup vs baseline: 1.0152x; 1.0009x over previous
import functools

import jax
import jax.numpy as jnp
from jax import lax
from jax.experimental import pallas as pl
from jax.experimental.pallas import tpu as pltpu
from jax.experimental.pallas import tpu_sc as plsc

D = 128
N = 10000
NPAD = 10240
E = 320000
G = 256
NPRED = 100
PPAD = 104
NC, NS = 2, 16
NW = NC * NS
EW = 10240
EPAD = EW * NW
K = 128
NCHUNK = EW // K
NPAIR = NCHUNK // 2
ROWS_PER_TILE = NPAD // NS
BN = 1024
NBLK = NPAD // BN

def _relu(v):
    return jnp.maximum(v, 0.0)


@functools.lru_cache(maxsize=None)
def _make_edge_pass(with_p, k, ew0=EW):
    ew1 = 2 * EW - ew0
    np0, np1 = ew0 // (2 * k), ew1 // (2 * k)
    scratch = [
        pltpu.VMEM_SHARED((NPAD, D), jnp.float32),
    ]
    for _ in range(2):
        scratch += [
            pltpu.VMEM((k,), jnp.int32),
            pltpu.VMEM((k,), jnp.int32),
            pltpu.VMEM((k, D), jnp.float32),
            pltpu.SemaphoreType.DMA,
            pltpu.SemaphoreType.DMA,
            pltpu.SemaphoreType.DMA,
        ]
        if with_p:
            scratch += [
                pltpu.VMEM((k,), jnp.int32),
                pltpu.VMEM((k, D), jnp.float32),
                pltpu.SemaphoreType.DMA,
            ]

    def body(*refs):
        nh = 5 if with_p else 3
        hbm = refs[:nh]
        out_h = refs[nh]
        sc = list(refs[nh + 1:])
        aggr_sh = sc.pop(0)
        nper = len(sc) // 2
        bufs = [sc[:nper], sc[nper:]]
        if with_p:
            attr_h, dst_h, t_h, src_h, p_h = hbm
        else:
            attr_h, dst_h, t_h = hbm
        c = lax.axis_index("c")
        s = lax.axis_index("s")
        npair = jnp.where(c == 0, np0, np1)
        ebase = jnp.where(c == 0, s * ew0, NS * ew0 + s * ew1)

        def unpack(b):
            if with_p:
                aidx, didx, rows, sem_i, sem_g, sem_s, sidx, prow, sem_p = \
                    bufs[b]
            else:
                aidx, didx, rows, sem_i, sem_g, sem_s = bufs[b]
                sidx = prow = sem_p = None
            return aidx, didx, rows, sem_i, sem_g, sem_s, sidx, prow, sem_p

        def start_idx(b, base):
            aidx, didx, _, sem_i, _, _, sidx, _, _ = unpack(b)
            pltpu.async_copy(attr_h.at[pl.ds(base, k)], aidx, sem_i)
            pltpu.async_copy(dst_h.at[pl.ds(base, k)], didx, sem_i)
            if with_p:
                pltpu.async_copy(src_h.at[pl.ds(base, k)], sidx, sem_i)

        def wait_idx_start_gathers(b, base):
            aidx, didx, rows, sem_i, sem_g, _, sidx, prow, sem_p = unpack(b)
            pltpu.make_async_copy(attr_h.at[pl.ds(base, k)], aidx,
                                  sem_i).wait()
            pltpu.make_async_copy(dst_h.at[pl.ds(base, k)], didx,
                                  sem_i).wait()
            if with_p:
                pltpu.make_async_copy(src_h.at[pl.ds(base, k)], sidx,
                                      sem_i).wait()
                pltpu.async_copy(p_h.at[sidx], prow, sem_p)
            pltpu.async_copy(t_h.at[aidx], rows, sem_g)

        def wait_gathers(b):
            aidx, _, rows, _, sem_g, _, sidx, prow, sem_p = unpack(b)
            pltpu.make_async_copy(t_h.at[aidx], rows, sem_g).wait()
            if with_p:
                pltpu.make_async_copy(p_h.at[sidx], prow, sem_p).wait()

        _, _, rows0, _, _, _, _, _, _ = unpack(0)

        def zrow(r, carry):
            for cc in range(D // 16):
                rows0[r, pl.ds(cc * 16, 16)] = jnp.zeros((16,), jnp.float32)
            return carry
        lax.fori_loop(0, k, zrow, 0)
        row0 = s * ROWS_PER_TILE
        for q in range(ROWS_PER_TILE // k):
            pltpu.sync_copy(rows0, aggr_sh.at[pl.ds(row0 + q * k, k)])
        plsc.subcore_barrier()
        start_idx(0, ebase)
        wait_idx_start_gathers(0, ebase)

        def pair(t, carry):
            for b in range(2):
                j2 = 2 * t + b
                nb = 1 - b
                (aidx, didx, rows, sem_i, sem_g, sem_s,
                 sidx, prow, sem_p) = unpack(b)
                n_bufs = unpack(nb)
                nbase = ebase + (j2 + 1) * k

                def wait_prev_scat():
                    pltpu.make_async_copy(
                        n_bufs[2], aggr_sh.at[n_bufs[1]], n_bufs[5]).wait()

                def launch():
                    wait_idx_start_gathers(nb, nbase)

                if b == 1:
                    wait_prev_scat()
                else:
                    pl.when(t > 0)(wait_prev_scat)
                if b == 0:
                    start_idx(nb, nbase)
                else:
                    pl.when(t < npair - 1)(lambda: start_idx(nb, nbase))
                wait_gathers(b)
                if with_p:
                    def rbody(r4, rc):
                        for u in range(4):
                            r = 4 * r4 + u
                            for cc in range(D // 16):
                                sl = pl.ds(cc * 16, 16)
                                rows[r, sl] = _relu(rows[r, sl]
                                                    + prow[r, sl])
                        return rc
                    lax.fori_loop(0, k // 4, rbody, 0)
                if b == 0:
                    launch()
                else:
                    pl.when(t < npair - 1)(launch)
                pltpu.async_copy(rows, aggr_sh.at[didx], sem_s, add=True)
            return carry
        lax.fori_loop(0, npair, pair, 0)
        b1 = unpack(1)
        pltpu.make_async_copy(b1[2], aggr_sh.at[b1[1]], b1[5]).wait()
        plsc.subcore_barrier()
        pltpu.sync_copy(aggr_sh.at[pl.ds(row0, ROWS_PER_TILE)],
                        out_h.at[c, pl.ds(row0, ROWS_PER_TILE)])

    mesh = plsc.VectorSubcoreMesh(
        core_axis_name="c", subcore_axis_name="s",
        num_cores=NC, num_subcores=NS)
    return pl.kernel(
        body,
        out_type=jax.ShapeDtypeStruct((NC, NPAD, D), jnp.float32),
        mesh=mesh,
        scratch_types=scratch,
    )


def _edge_pass_l0(*args):
    return _make_edge_pass(False, 128, 17408)(*args)


def _edge_pass_l1(*args):
    return _make_edge_pass(True, 80, 16000)(*args)


def _prep_tables(relp, w0b, b0, w1b, b1):
    def body(r_ref, w0_ref, b0_ref, w1_ref, b1_ref, t0_ref, t1_ref):
        r = r_ref[...]
        t0_ref[...] = _relu(
            jnp.dot(r, w0_ref[...], preferred_element_type=jnp.float32)
            + b0_ref[...])
        t1_ref[...] = (
            jnp.dot(r, w1_ref[...], preferred_element_type=jnp.float32)
            + b1_ref[...])
    return pl.pallas_call(
        body,
        out_shape=(jax.ShapeDtypeStruct((PPAD, D), jnp.float32),
                   jax.ShapeDtypeStruct((PPAD, D), jnp.float32)),
    )(relp, w0b, b0, w1b, b1)


def _node_update(parts, h, wa, wb, cb, gw, gb, batch_col):
    with_h = h is not None

    def body(*refs):
        if with_h:
            (parts_ref, h_ref, wa_ref, wb_ref, cb_ref, gw_ref, gb_ref,
             bi_ref, hn_ref, gate_ref, gmax_ref) = refs
        else:
            (parts_ref, wb_ref, cb_ref, gw_ref, gb_ref,
             bi_ref, hn_ref, gate_ref, gmax_ref) = refs
        i = pl.program_id(0)
        aggr = parts_ref[0] + parts_ref[1]
        z = jnp.dot(aggr, wb_ref[...], preferred_element_type=jnp.float32)
        z = z + cb_ref[...]
        if with_h:
            hb = h_ref[...]
            z = z + jnp.dot(hb, wa_ref[...],
                            preferred_element_type=jnp.float32)
            hn = hb + _relu(z)
        else:
            hn = _relu(z)
        hn_ref[...] = hn
        gate = (jnp.dot(hn, gw_ref[...], preferred_element_type=jnp.float32)
                + gb_ref[...])
        gate_ref[...] = gate
        mask = bi_ref[...] == lax.broadcasted_iota(jnp.int32, (BN, G), 1)
        bmax = jnp.max(jnp.where(mask, gate, -jnp.inf), axis=0, keepdims=True)

        @pl.when(i == 0)
        def _():
            gmax_ref[...] = jnp.full((1, G), -jnp.inf, jnp.float32)
        gmax_ref[...] = jnp.maximum(gmax_ref[...], bmax)

    full = lambda shape: pl.BlockSpec(shape, lambda i: (0,) * len(shape))
    in_specs = [pl.BlockSpec((NC, BN, D), lambda i: (0, i, 0))]
    args = [parts]
    if with_h:
        in_specs.append(pl.BlockSpec((BN, D), lambda i: (i, 0)))
        args.append(h)
        in_specs.append(full((D, D)))
        args.append(wa)
    in_specs += [full((D, D)), full((1, D)), full((D, 1)), full((1, 1)),
                 pl.BlockSpec((BN, 1), lambda i: (i, 0))]
    args += [wb, cb, gw, gb, batch_col]
    return pl.pallas_call(
        body,
        grid=(NBLK,),
        in_specs=in_specs,
        out_specs=(pl.BlockSpec((BN, D), lambda i: (i, 0)),
                   pl.BlockSpec((BN, 1), lambda i: (i, 0)),
                   pl.BlockSpec((1, G), lambda i: (0, 0))),
        out_shape=(jax.ShapeDtypeStruct((NPAD, D), jnp.float32),
                   jax.ShapeDtypeStruct((NPAD, 1), jnp.float32),
                   jax.ShapeDtypeStruct((1, G), jnp.float32)),
    )(*args)


def _pool(hn, gate, gmax, batch_col, aw, ab, pw, g, gwa, gwb, glb):
    with_p = pw is not None
    with_g = g is not None

    def body(*refs):
        refs = list(refs)
        h_ref, gate_ref, gmax_ref, bi_ref, aw_ref, ab_ref = refs[:6]
        rest = refs[6:]
        pw_ref = rest.pop(0) if with_p else None
        g_ref = rest.pop(0) if with_g else None
        gwa_ref, gwb_ref, glb_ref = rest[:3]
        outs = rest[3:]
        glnum_ref, denom_ref, gout_ref = outs[:3]
        p_ref = outs[3] if with_p else None
        i = pl.program_id(0)
        gm = gmax_ref[...]
        gmf = jnp.where(jnp.isfinite(gm), gm, 0.0)
        bi = bi_ref[...]
        mask = bi == lax.broadcasted_iota(jnp.int32, (BN, G), 1)
        ohf = mask.astype(jnp.float32)
        gnode = jnp.sum(jnp.where(mask, gmf, 0.0), axis=1, keepdims=True)
        egate = jnp.where(bi < G, jnp.exp(gate_ref[...] - gnode), 0.0)
        hb = h_ref[...]
        xt = (jnp.dot(hb, aw_ref[...], preferred_element_type=jnp.float32)
              + ab_ref[...])
        gn = lax.dot_general(ohf, egate * xt, (((0,), (0,)), ((), ())),
                             preferred_element_type=jnp.float32)
        dn = lax.dot_general(ohf, egate, (((0,), (0,)), ((), ())),
                             preferred_element_type=jnp.float32)

        @pl.when(i == 0)
        def _():
            glnum_ref[...] = jnp.zeros((G, D), jnp.float32)
            denom_ref[...] = jnp.zeros((G, 1), jnp.float32)
        glnum_ref[...] += gn
        denom_ref[...] += dn
        if with_p:
            p_ref[...] = jnp.dot(hb, pw_ref[...],
                                 preferred_element_type=jnp.float32)

        @pl.when(i == NBLK - 1)
        def _():
            gl = glnum_ref[...] / (denom_ref[...] + 1e-16)
            z = jnp.dot(gl, gwa_ref[...], preferred_element_type=jnp.float32)
            z = z + glb_ref[...]
            if with_g:
                gprev = g_ref[...]
                z = z + jnp.dot(gprev, gwb_ref[...],
                                preferred_element_type=jnp.float32)
                gout_ref[...] = gprev + _relu(z)
            else:
                gout_ref[...] = _relu(z)

    full = lambda shape: pl.BlockSpec(shape, lambda i: (0,) * len(shape))
    in_specs = [pl.BlockSpec((BN, D), lambda i: (i, 0)),
                pl.BlockSpec((BN, 1), lambda i: (i, 0)),
                full((1, G)),
                pl.BlockSpec((BN, 1), lambda i: (i, 0)),
                full((D, D)), full((1, D))]
    args = [hn, gate, gmax, batch_col, aw, ab]
    if with_p:
        in_specs.append(full((D, D)))
        args.append(pw)
    if with_g:
        in_specs.append(full((G, D)))
        args.append(g)
    in_specs += [full((D, D)), full((D, D)) if with_g else full((D, D)),
                 full((1, D))]
    args += [gwa, gwb if with_g else gwa, glb]
    out_specs = [pl.BlockSpec((G, D), lambda i: (0, 0)),
                 pl.BlockSpec((G, 1), lambda i: (0, 0)),
                 pl.BlockSpec((G, D), lambda i: (0, 0))]
    out_shape = [jax.ShapeDtypeStruct((G, D), jnp.float32),
                 jax.ShapeDtypeStruct((G, 1), jnp.float32),
                 jax.ShapeDtypeStruct((G, D), jnp.float32)]
    if with_p:
        out_specs.append(pl.BlockSpec((BN, D), lambda i: (i, 0)))
        out_shape.append(jax.ShapeDtypeStruct((NPAD, D), jnp.float32))
    res = pl.pallas_call(
        body,
        grid=(NBLK,),
        in_specs=in_specs,
        out_specs=tuple(out_specs),
        out_shape=tuple(out_shape),
    )(*args)
    return (res[2], res[3]) if with_p else (res[2],)


def kernel(params, x, edge_index, edge_attr, batch_idx):
    src = edge_index[0].astype(jnp.int32)
    dst = edge_index[1].astype(jnp.int32)
    attr = edge_attr.astype(jnp.int32)
    batch = batch_idx.astype(jnp.int32)

    pad_e = EPAD - E
    src_p = jnp.concatenate([src, jnp.zeros((pad_e,), jnp.int32)])
    dst_p = jnp.concatenate([dst, jnp.full((pad_e,), N, jnp.int32)])
    attr_p = jnp.concatenate([attr, jnp.zeros((pad_e,), jnp.int32)])
    batch_col = jnp.concatenate(
        [batch, jnp.full((NPAD - N,), G, jnp.int32)]).reshape(NPAD, 1)

    relp = jnp.pad(params['rel_table'], ((0, PPAD - NPRED), (0, 0)))
    row = lambda b: b.reshape(1, -1)

    msg_W0, msg_b0 = params['msg_W0'], params['msg_b0']
    msg_W1, msg_b1 = params['msg_W1'], params['msg_b1']
    t0r, t1 = _prep_tables(relp, msg_W0[D:], row(msg_b0),
                           msg_W1[D:], row(msg_b1))

    parts0 = _edge_pass_l0(attr_p, dst_p, t0r)
    h1, gate0, gmax0 = _node_update(
        parts0, None, None, params['comb_W0'][D:], row(params['comb_b0']),
        params['gate_W0'], row(params['gate_b0']), batch_col)
    g1, p1 = _pool(h1, gate0, gmax0, batch_col,
                   params['attn_W0'], row(params['attn_b0']), msg_W1[:D],
                   None, params['glob_W0'][:D], None,
                   row(params['glob_b0']))

    parts1 = _edge_pass_l1(attr_p, dst_p, t1, src_p, p1)
    h2, gate1, gmax1 = _node_update(
        parts1, h1, params['comb_W1'][:D], params['comb_W1'][D:],
        row(params['comb_b1']), params['gate_W1'], row(params['gate_b1']),
        batch_col)
    (g2,) = _pool(h2, gate1, gmax1, batch_col,
                  params['attn_W1'], row(params['attn_b1']), None,
                  g1, params['glob_W1'][:D], params['glob_W1'][D:],
                  row(params['glob_b1']))

    return h2[:N], g2
